# Initial kernel scaffold; baseline (speedup 1.0000x reference)
import jax
import jax.numpy as jnp
from jax.experimental import pallas as pl


def kernel(node_feats, edge_feats, W_node, b_node, W_edge, b_edge, edge_index):
    raise NotImplementedError("write your pallas kernel here")


# speedup vs baseline: 1.7318x; 1.7318x over previous
import functools

import jax
import jax.numpy as jnp
from jax import lax
from jax.experimental import pallas as pl
from jax.experimental.pallas import tpu as pltpu
from jax.experimental.pallas import tpu_sc as plsc

D_NODE = 128
D_EDGE = 16
D_OUT = D_NODE + D_EDGE
PACK = D_NODE // D_EDGE


def _proj_kernel(x_ref, wt_ref, b_ref, o_ref):
    o_ref[...] = (
        jnp.dot(x_ref[...], wt_ref[...], preferred_element_type=jnp.float32)
        + b_ref[...]
    )


def _project(x, wt, b, block_rows):
    n = x.shape[0]
    d_in = x.shape[1]
    d_out = wt.shape[1]
    grid = n // block_rows
    return pl.pallas_call(
        _proj_kernel,
        grid=(grid,),
        in_specs=[
            pl.BlockSpec((block_rows, d_in), lambda i: (i, 0)),
            pl.BlockSpec((d_in, d_out), lambda i: (0, 0)),
            pl.BlockSpec((1, d_out), lambda i: (0, 0)),
        ],
        out_specs=pl.BlockSpec((block_rows, d_out), lambda i: (i, 0)),
        out_shape=jax.ShapeDtypeStruct((n, d_out), jnp.float32),
    )(x, wt, b.reshape(1, d_out))


def _make_sc_assemble(n_edges, chunk):
    n_workers = 32
    per_w = n_edges // n_workers
    n_chunks = per_w // chunk
    mesh = plsc.VectorSubcoreMesh(core_axis_name="c", subcore_axis_name="s")

    @functools.partial(
        pl.kernel,
        mesh=mesh,
        out_type=jax.ShapeDtypeStruct((n_edges, D_OUT), jnp.float32),
        scratch_types=[
            pltpu.VMEM((chunk,), jnp.int32),
            pltpu.VMEM((chunk, D_NODE), jnp.float32),
            pltpu.VMEM((chunk, D_EDGE), jnp.float32),
            pltpu.SemaphoreType.DMA,
        ],
    )
    def sc_assemble(node_h_hbm, edge_h_hbm, src_hbm, out_hbm, idx_v, rows_v, eh_v, sem):
        wid = lax.axis_index("s") * 2 + lax.axis_index("c")
        base = wid * per_w

        def body(i, carry):
            off = base + i * chunk
            pltpu.sync_copy(src_hbm.at[pl.ds(off, chunk)], idx_v)
            pltpu.sync_copy(edge_h_hbm.at[pl.ds(off, chunk)], eh_v)
            pltpu.async_copy(node_h_hbm.at[idx_v], rows_v, sem).wait()
            pltpu.sync_copy(rows_v, out_hbm.at[pl.ds(off, chunk), pl.ds(0, D_NODE)])
            pltpu.sync_copy(eh_v, out_hbm.at[pl.ds(off, chunk), pl.ds(D_NODE, D_EDGE)])
            return carry

        lax.fori_loop(0, n_chunks, body, 0)

    return sc_assemble


def kernel(node_feats, edge_feats, W_node, b_node, W_edge, b_edge, edge_index):
    n_edges = edge_feats.shape[0]

    node_h = _project(node_feats, W_node.T, b_node, block_rows=2000)

    wt_big = jax.scipy.linalg.block_diag(*([W_edge.T] * PACK))
    b_big = jnp.tile(b_edge, PACK)
    ef_packed = edge_feats.reshape(n_edges // PACK, D_NODE)
    edge_h = _project(ef_packed, wt_big, b_big, block_rows=5000).reshape(
        n_edges, D_EDGE
    )

    src = jnp.asarray(edge_index[0], dtype=jnp.int32)
    return _make_sc_assemble(n_edges, chunk=400)(node_h, edge_h, src)
